# baseline (speedup 1.0000x reference)
import functools

import jax
import jax.numpy as jnp
from jax import lax
from jax.experimental import pallas as pl
from jax.experimental.pallas import tpu as pltpu
from jax.experimental.pallas import tpu_sc as plsc

_N = 16384
_D = 1024
_K = 8
_EPS = 1e-05
_L = 16
_C = _D // _L
_NW = 32
_S = 2
_NS = _N // _S
_RPW = _NS // _NW
_B = 16
_NBLK = _RPW // _B
_BLK1 = 2048


def _label_body(x_ref, c_ref, rec_ref):
    xb = x_ref[...]
    c = c_ref[...]
    dots = lax.dot_general(
        xb, c, (((1,), (1,)), ((), ())), preferred_element_type=jnp.float32
    )
    c2 = jnp.sum(c * c, axis=1)
    dist = c2[None, :] - 2.0 * dots
    lbl = jnp.argmin(dist, axis=1).astype(jnp.float32)
    mean = jnp.mean(xb, axis=1)
    xc = xb - mean[:, None]
    var = jnp.sum(xc * xc, axis=1) * (1.0 / (_D - 1))
    rinv = lax.rsqrt(var + _EPS)
    pad = jnp.zeros((_BLK1, 5), jnp.float32)
    rec_ref[...] = jnp.concatenate(
        [lbl[:, None], mean[:, None], rinv[:, None], pad], axis=1
    )


def _labels(x, centroids, s):
    off = s * (_NS // _BLK1)
    return pl.pallas_call(
        _label_body,
        grid=(_NS // _BLK1,),
        in_specs=[
            pl.BlockSpec((_BLK1, _D), lambda i: (off + i, 0)),
            pl.BlockSpec((_K, _D), lambda i: (0, 0)),
        ],
        out_specs=pl.BlockSpec((_BLK1, 8), lambda i: (i, 0)),
        out_shape=jax.ShapeDtypeStruct((_NS, 8), jnp.float32),
    )(x, centroids)


def _tanh(x):
    t = jnp.exp(-2.0 * jnp.abs(x))
    r = (1.0 - t) / (1.0 + t)
    return jnp.where(x < 0, -r, r)


def _sigmoid(x):
    t = jnp.exp(-jnp.abs(x))
    return jnp.where(x >= 0, 1.0, t) / (1.0 + t)


def _log1p01(t):
    s = t / (2.0 + t)
    s2 = s * s
    p = 1.0 / 11.0
    p = p * s2 + 1.0 / 9.0
    p = p * s2 + 1.0 / 7.0
    p = p * s2 + 1.0 / 5.0
    p = p * s2 + 1.0 / 3.0
    p = p * s2 + 1.0
    return 2.0 * s * p


def _act_relu(x):
    return jnp.maximum(x, 0.0)


def _act_gelu(x):
    u = 0.7978845608028654 * (x + 0.044715 * (x * x * x))
    return 0.5 * x * (1.0 + _tanh(u))


def _act_silu(x):
    return x * _sigmoid(x)


def _act_relu6(x):
    return jnp.minimum(jnp.maximum(x, 0.0), 6.0)


def _act_elu(x):
    return jnp.where(x > 0, x, jnp.exp(jnp.minimum(x, 0.0)) - 1.0)


def _act_softplus(x):
    return jnp.maximum(x, 0.0) + _log1p01(jnp.exp(-jnp.abs(x)))


_ACT_FNS = [_act_relu, _act_gelu, _tanh, _act_silu, _sigmoid, _act_relu6,
            _act_elu, _act_softplus]


def _sc_body(s, x_hbm, rec_hbm, o_hbm,
             rec_v, in0, in1, out0, out1, sin0, sin1, sout0, sout1):
    wid = lax.axis_index("s") * 2 + lax.axis_index("c")
    row0 = s * _NS + wid * _RPW
    orow0 = wid * _RPW
    pltpu.sync_copy(
        rec_hbm.at[pl.ds(wid * (_RPW * 8), _RPW * 8)],
        rec_v.at[pl.ds(0, _RPW * 8)],
    )

    ins = (in0, in1)
    outs = (out0, out1)
    sins = (sin0, sin1)
    souts = (sout0, sout1)

    def xslice(b):
        return x_hbm.at[pl.ds(row0 + b * _B, _B), :]

    def oslice(b):
        return o_hbm.at[pl.ds(orow0 + b * _B, _B), :]

    def process(b, in_v, out_v):
        def row_body(r, _):
            gro = b * _B + r
            rv = rec_v[pl.ds(pl.multiple_of(gro * 8, 8), _L)]
            lblf = rv[0]
            rinv = rv[2]
            bias = -rv[1] * rinv

            def leaf(actfn):
                def run():
                    @plsc.parallel_loop(0, _C, 1, unroll=8)
                    def chunk(i):
                        c0 = pl.multiple_of(i * _L, 8)
                        xv = in_v[r, pl.ds(c0, _L)]
                        out_v[r, pl.ds(c0, _L)] = actfn(xv * rinv + bias)
                    return 0
                return run

            leaves = [leaf(f) for f in _ACT_FNS]
            lax.cond(
                lblf < 4.0,
                lambda: lax.cond(
                    lblf < 2.0,
                    lambda: lax.cond(lblf < 1.0, leaves[0], leaves[1]),
                    lambda: lax.cond(lblf < 3.0, leaves[2], leaves[3]),
                ),
                lambda: lax.cond(
                    lblf < 6.0,
                    lambda: lax.cond(lblf < 5.0, leaves[4], leaves[5]),
                    lambda: lax.cond(lblf < 7.0, leaves[6], leaves[7]),
                ),
            )
            return 0

        lax.fori_loop(0, _B, row_body, 0)

    pltpu.async_copy(xslice(0), in0, sin0)
    pltpu.async_copy(xslice(1), in1, sin1)

    def pair_body(p, _):
        for c in range(2):
            b = 2 * p + c
            pltpu.make_async_copy(xslice(b), ins[c], sins[c]).wait()
            lax.cond(
                p > 0,
                lambda c=c, b=b: pltpu.make_async_copy(
                    outs[c], oslice(b - 2), souts[c]).wait() or 0,
                lambda: 0,
            )
            process(b, ins[c], outs[c])
            pltpu.async_copy(outs[c], oslice(b), souts[c])
            lax.cond(
                p < _NBLK // 2 - 1,
                lambda c=c, b=b: pltpu.async_copy(
                    xslice(b + 2), ins[c], sins[c]) and 0,
                lambda: 0,
            )
        return 0

    lax.fori_loop(0, _NBLK // 2, pair_body, 0)
    pltpu.make_async_copy(out0, oslice(_NBLK - 2), sout0).wait()
    pltpu.make_async_copy(out1, oslice(_NBLK - 1), sout1).wait()


def _make_sc_call(s):
    return functools.partial(
        pl.kernel,
        mesh=plsc.VectorSubcoreMesh(core_axis_name="c", subcore_axis_name="s"),
        compiler_params=pltpu.CompilerParams(
            needs_layout_passes=False, use_tc_tiling_on_sc=True
        ),
        out_type=jax.ShapeDtypeStruct((_NS, _D), jnp.float32),
        scratch_types=[
            pltpu.VMEM((_RPW * 8 + _L,), jnp.float32),
            pltpu.VMEM((_B, _D), jnp.float32),
            pltpu.VMEM((_B, _D), jnp.float32),
            pltpu.VMEM((_B, _D), jnp.float32),
            pltpu.VMEM((_B, _D), jnp.float32),
            pltpu.SemaphoreType.DMA,
            pltpu.SemaphoreType.DMA,
            pltpu.SemaphoreType.DMA,
            pltpu.SemaphoreType.DMA,
        ],
    )(functools.partial(_sc_body, s))


_sc_calls = [_make_sc_call(s) for s in range(_S)]


@jax.jit
def kernel(x, centroids):
    outs = []
    for s in range(_S):
        rec = _labels(x, centroids, s)
        outs.append(_sc_calls[s](x, rec.reshape(-1)))
    return jnp.concatenate(outs, axis=0)


# speedup vs baseline: 1.1818x; 1.1818x over previous
import functools

import jax
import jax.numpy as jnp
from jax import lax
from jax.experimental import pallas as pl
from jax.experimental.pallas import tpu as pltpu
from jax.experimental.pallas import tpu_sc as plsc

_N = 16384
_D = 1024
_K = 8
_EPS = 1e-05
_L = 16
_C = _D // _L
_NW = 32
_RPW = _N // _NW
_B = 16
_NBLK = _RPW // _B
_BLK1 = 2048


def _label_body(x_ref, c_ref, rec_ref):
    xb = x_ref[...]
    c = c_ref[...]
    dots = lax.dot_general(
        xb, c, (((1,), (1,)), ((), ())), preferred_element_type=jnp.float32
    )
    c2 = jnp.sum(c * c, axis=1)
    dist = c2[None, :] - 2.0 * dots
    lbl = jnp.argmin(dist, axis=1).astype(jnp.float32)
    ssum = jnp.sum(xb, axis=1)
    qsum = jnp.sum(xb * xb, axis=1)
    mean = ssum * (1.0 / _D)
    var = (qsum - ssum * mean) * (1.0 / (_D - 1))
    rinv = lax.rsqrt(var + _EPS)
    pad = jnp.zeros((_BLK1, 5), jnp.float32)
    rec_ref[...] = jnp.concatenate(
        [lbl[:, None], mean[:, None], rinv[:, None], pad], axis=1
    )


def _labels(x, centroids):
    return pl.pallas_call(
        _label_body,
        grid=(_N // _BLK1,),
        in_specs=[
            pl.BlockSpec((_BLK1, _D), lambda i: (i, 0)),
            pl.BlockSpec((_K, _D), lambda i: (0, 0)),
        ],
        out_specs=pl.BlockSpec((_BLK1, 8), lambda i: (i, 0)),
        out_shape=jax.ShapeDtypeStruct((_N, 8), jnp.float32),
    )(x, centroids)


def _tanh(x):
    t = jnp.exp(-2.0 * jnp.abs(x))
    r = (1.0 - t) / (1.0 + t)
    return jnp.where(x < 0, -r, r)


def _sigmoid(x):
    t = jnp.exp(-jnp.abs(x))
    return jnp.where(x >= 0, 1.0, t) / (1.0 + t)


def _log1p01(t):
    s = t / (2.0 + t)
    s2 = s * s
    p = 1.0 / 11.0
    p = p * s2 + 1.0 / 9.0
    p = p * s2 + 1.0 / 7.0
    p = p * s2 + 1.0 / 5.0
    p = p * s2 + 1.0 / 3.0
    p = p * s2 + 1.0
    return 2.0 * s * p


def _act_relu(x):
    return jnp.maximum(x, 0.0)


def _act_gelu(x):
    u = 0.7978845608028654 * (x + 0.044715 * (x * x * x))
    return 0.5 * x * (1.0 + _tanh(u))


def _act_silu(x):
    return x * _sigmoid(x)


def _act_relu6(x):
    return jnp.minimum(jnp.maximum(x, 0.0), 6.0)


def _act_elu(x):
    return jnp.where(x > 0, x, jnp.exp(jnp.minimum(x, 0.0)) - 1.0)


def _act_softplus(x):
    return jnp.maximum(x, 0.0) + _log1p01(jnp.exp(-jnp.abs(x)))


_ACT_FNS = [_act_relu, _act_gelu, _tanh, _act_silu, _sigmoid, _act_relu6,
            _act_elu, _act_softplus]


def _sc_body(x_hbm, rec_hbm, o_hbm,
             rec_v, in0, in1, out0, out1, sin0, sin1, sout0, sout1):
    wid = lax.axis_index("s") * 2 + lax.axis_index("c")
    row0 = wid * _RPW
    pltpu.sync_copy(
        rec_hbm.at[pl.ds(wid * (_RPW * 8), _RPW * 8)],
        rec_v.at[pl.ds(0, _RPW * 8)],
    )

    ins = (in0, in1)
    outs = (out0, out1)
    sins = (sin0, sin1)
    souts = (sout0, sout1)

    def xslice(b):
        return x_hbm.at[pl.ds(row0 + b * _B, _B), :]

    def oslice(b):
        return o_hbm.at[pl.ds(row0 + b * _B, _B), :]

    def process(b, in_v, out_v):
        def row_body(r, _):
            gro = b * _B + r
            rv = rec_v[pl.ds(pl.multiple_of(gro * 8, 8), _L)]
            lblf = rv[0]
            rinv = rv[2]
            bias = -rv[1] * rinv

            def leaf(actfn):
                def run():
                    @plsc.parallel_loop(0, _C, 1, unroll=8)
                    def chunk(i):
                        c0 = pl.multiple_of(i * _L, 8)
                        xv = in_v[r, pl.ds(c0, _L)]
                        out_v[r, pl.ds(c0, _L)] = actfn(xv * rinv + bias)
                    return 0
                return run

            leaves = [leaf(f) for f in _ACT_FNS]
            lax.cond(
                lblf < 4.0,
                lambda: lax.cond(
                    lblf < 2.0,
                    lambda: lax.cond(lblf < 1.0, leaves[0], leaves[1]),
                    lambda: lax.cond(lblf < 3.0, leaves[2], leaves[3]),
                ),
                lambda: lax.cond(
                    lblf < 6.0,
                    lambda: lax.cond(lblf < 5.0, leaves[4], leaves[5]),
                    lambda: lax.cond(lblf < 7.0, leaves[6], leaves[7]),
                ),
            )
            return 0

        lax.fori_loop(0, _B, row_body, 0)

    pltpu.async_copy(xslice(0), in0, sin0)
    pltpu.async_copy(xslice(1), in1, sin1)

    def pair_body(p, _):
        for c in range(2):
            b = 2 * p + c
            pltpu.make_async_copy(xslice(b), ins[c], sins[c]).wait()
            lax.cond(
                p > 0,
                lambda c=c, b=b: pltpu.make_async_copy(
                    outs[c], oslice(b - 2), souts[c]).wait() or 0,
                lambda: 0,
            )
            process(b, ins[c], outs[c])
            pltpu.async_copy(outs[c], oslice(b), souts[c])
            lax.cond(
                p < _NBLK // 2 - 1,
                lambda c=c, b=b: pltpu.async_copy(
                    xslice(b + 2), ins[c], sins[c]) and 0,
                lambda: 0,
            )
        return 0

    lax.fori_loop(0, _NBLK // 2, pair_body, 0)
    pltpu.make_async_copy(out0, oslice(_NBLK - 2), sout0).wait()
    pltpu.make_async_copy(out1, oslice(_NBLK - 1), sout1).wait()


_sc_call = functools.partial(
    pl.kernel,
    mesh=plsc.VectorSubcoreMesh(core_axis_name="c", subcore_axis_name="s"),
    compiler_params=pltpu.CompilerParams(
        needs_layout_passes=False, use_tc_tiling_on_sc=True
    ),
    out_type=jax.ShapeDtypeStruct((_N, _D), jnp.float32),
    scratch_types=[
        pltpu.VMEM((_RPW * 8 + _L,), jnp.float32),
        pltpu.VMEM((_B, _D), jnp.float32),
        pltpu.VMEM((_B, _D), jnp.float32),
        pltpu.VMEM((_B, _D), jnp.float32),
        pltpu.VMEM((_B, _D), jnp.float32),
        pltpu.SemaphoreType.DMA,
        pltpu.SemaphoreType.DMA,
        pltpu.SemaphoreType.DMA,
        pltpu.SemaphoreType.DMA,
    ],
)(_sc_body)


@jax.jit
def kernel(x, centroids):
    rec = _labels(x, centroids)
    return _sc_call(x, rec.reshape(-1))


# speedup vs baseline: 1.1832x; 1.0012x over previous
import functools

import jax
import jax.numpy as jnp
from jax import lax
from jax.experimental import pallas as pl
from jax.experimental.pallas import tpu as pltpu
from jax.experimental.pallas import tpu_sc as plsc

_N = 16384
_D = 1024
_K = 8
_EPS = 1e-05
_L = 16
_C = _D // _L
_NW = 32
_RPW = _N // _NW
_B = 16
_NBLK = _RPW // _B
_BLK1 = 1024


def _label_body(x_ref, c_ref, rec_ref):
    xb = x_ref[...]
    c = c_ref[...]
    dots = lax.dot_general(
        xb, c, (((1,), (1,)), ((), ())), preferred_element_type=jnp.float32
    )
    c2 = jnp.sum(c * c, axis=1)
    dist = c2[None, :] - 2.0 * dots
    lbl = jnp.argmin(dist, axis=1).astype(jnp.float32)
    ssum = jnp.sum(xb, axis=1)
    qsum = jnp.sum(xb * xb, axis=1)
    mean = ssum * (1.0 / _D)
    var = (qsum - ssum * mean) * (1.0 / (_D - 1))
    rinv = lax.rsqrt(var + _EPS)
    pad = jnp.zeros((_BLK1, 5), jnp.float32)
    rec_ref[...] = jnp.concatenate(
        [lbl[:, None], mean[:, None], rinv[:, None], pad], axis=1
    )


def _labels(x, centroids):
    return pl.pallas_call(
        _label_body,
        grid=(_N // _BLK1,),
        in_specs=[
            pl.BlockSpec((_BLK1, _D), lambda i: (i, 0)),
            pl.BlockSpec((_K, _D), lambda i: (0, 0)),
        ],
        out_specs=pl.BlockSpec((_BLK1, 8), lambda i: (i, 0)),
        out_shape=jax.ShapeDtypeStruct((_N, 8), jnp.float32),
    )(x, centroids)


def _tanh(x):
    t = jnp.exp(-2.0 * jnp.abs(x))
    r = (1.0 - t) / (1.0 + t)
    return jnp.where(x < 0, -r, r)


def _sigmoid(x):
    t = jnp.exp(-jnp.abs(x))
    return jnp.where(x >= 0, 1.0, t) / (1.0 + t)


def _log1p01(t):
    s = t / (2.0 + t)
    s2 = s * s
    p = 1.0 / 11.0
    p = p * s2 + 1.0 / 9.0
    p = p * s2 + 1.0 / 7.0
    p = p * s2 + 1.0 / 5.0
    p = p * s2 + 1.0 / 3.0
    p = p * s2 + 1.0
    return 2.0 * s * p


def _act_relu(x):
    return jnp.maximum(x, 0.0)


def _act_gelu(x):
    u = 0.7978845608028654 * (x + 0.044715 * (x * x * x))
    return 0.5 * x * (1.0 + _tanh(u))


def _act_silu(x):
    return x * _sigmoid(x)


def _act_relu6(x):
    return jnp.minimum(jnp.maximum(x, 0.0), 6.0)


def _act_elu(x):
    return jnp.where(x > 0, x, jnp.exp(jnp.minimum(x, 0.0)) - 1.0)


def _act_softplus(x):
    return jnp.maximum(x, 0.0) + _log1p01(jnp.exp(-jnp.abs(x)))


_ACT_FNS = [_act_relu, _act_gelu, _tanh, _act_silu, _sigmoid, _act_relu6,
            _act_elu, _act_softplus]


def _sc_body(x_hbm, rec_hbm, o_hbm,
             rec_v, in0, in1, out0, out1, sin0, sin1, sout0, sout1):
    wid = lax.axis_index("s") * 2 + lax.axis_index("c")
    row0 = wid * _RPW
    pltpu.sync_copy(
        rec_hbm.at[pl.ds(wid * (_RPW * 8), _RPW * 8)],
        rec_v.at[pl.ds(0, _RPW * 8)],
    )

    ins = (in0, in1)
    outs = (out0, out1)
    sins = (sin0, sin1)
    souts = (sout0, sout1)

    def xslice(b):
        return x_hbm.at[pl.ds(row0 + b * _B, _B), :]

    def oslice(b):
        return o_hbm.at[pl.ds(row0 + b * _B, _B), :]

    def process(b, in_v, out_v):
        def row_body(r, _):
            gro = b * _B + r
            rv = rec_v[pl.ds(pl.multiple_of(gro * 8, 8), _L)]
            lblf = rv[0]
            rinv = rv[2]
            bias = -rv[1] * rinv

            def leaf(actfn):
                def run():
                    @plsc.parallel_loop(0, _C, 1, unroll=8)
                    def chunk(i):
                        c0 = pl.multiple_of(i * _L, 8)
                        xv = in_v[r, pl.ds(c0, _L)]
                        out_v[r, pl.ds(c0, _L)] = actfn(xv * rinv + bias)
                    return 0
                return run

            leaves = [leaf(f) for f in _ACT_FNS]
            lax.cond(
                lblf < 4.0,
                lambda: lax.cond(
                    lblf < 2.0,
                    lambda: lax.cond(lblf < 1.0, leaves[0], leaves[1]),
                    lambda: lax.cond(lblf < 3.0, leaves[2], leaves[3]),
                ),
                lambda: lax.cond(
                    lblf < 6.0,
                    lambda: lax.cond(lblf < 5.0, leaves[4], leaves[5]),
                    lambda: lax.cond(lblf < 7.0, leaves[6], leaves[7]),
                ),
            )
            return 0

        lax.fori_loop(0, _B, row_body, 0)

    pltpu.async_copy(xslice(0), in0, sin0)
    pltpu.async_copy(xslice(1), in1, sin1)

    def pair_body(p, _):
        for c in range(2):
            b = 2 * p + c
            pltpu.make_async_copy(xslice(b), ins[c], sins[c]).wait()
            lax.cond(
                p > 0,
                lambda c=c, b=b: pltpu.make_async_copy(
                    outs[c], oslice(b - 2), souts[c]).wait() or 0,
                lambda: 0,
            )
            process(b, ins[c], outs[c])
            pltpu.async_copy(outs[c], oslice(b), souts[c])
            lax.cond(
                p < _NBLK // 2 - 1,
                lambda c=c, b=b: pltpu.async_copy(
                    xslice(b + 2), ins[c], sins[c]) and 0,
                lambda: 0,
            )
        return 0

    lax.fori_loop(0, _NBLK // 2, pair_body, 0)
    pltpu.make_async_copy(out0, oslice(_NBLK - 2), sout0).wait()
    pltpu.make_async_copy(out1, oslice(_NBLK - 1), sout1).wait()


_sc_call = functools.partial(
    pl.kernel,
    mesh=plsc.VectorSubcoreMesh(core_axis_name="c", subcore_axis_name="s"),
    compiler_params=pltpu.CompilerParams(
        needs_layout_passes=False, use_tc_tiling_on_sc=True
    ),
    out_type=jax.ShapeDtypeStruct((_N, _D), jnp.float32),
    scratch_types=[
        pltpu.VMEM((_RPW * 8 + _L,), jnp.float32),
        pltpu.VMEM((_B, _D), jnp.float32),
        pltpu.VMEM((_B, _D), jnp.float32),
        pltpu.VMEM((_B, _D), jnp.float32),
        pltpu.VMEM((_B, _D), jnp.float32),
        pltpu.SemaphoreType.DMA,
        pltpu.SemaphoreType.DMA,
        pltpu.SemaphoreType.DMA,
        pltpu.SemaphoreType.DMA,
    ],
)(_sc_body)


@jax.jit
def kernel(x, centroids):
    rec = _labels(x, centroids)
    return _sc_call(x, rec.reshape(-1))
